# baseline (speedup 1.0000x reference)
import functools

import jax
import jax.numpy as jnp
from jax import lax
from jax.experimental import pallas as pl
from jax.experimental.pallas import tpu as pltpu
from jax.experimental.pallas import tpu_sc as plsc


def _main_kernel(x8_ref, wg_ref, xt_ref, w1_ref, w2_ref, w3_ref,
                 sw1_ref, sw2_ref, sw3_ref, out_ref, lg_ref, yall_ref,
                 *, E, TILE):
    s = pl.program_id(0)
    x8 = x8_ref[...].astype(jnp.bfloat16)
    h1 = jnp.dot(x8, w1_ref[0].astype(jnp.bfloat16),
                 preferred_element_type=jnp.float32)
    h3 = jnp.dot(x8, w3_ref[0].astype(jnp.bfloat16),
                 preferred_element_type=jnp.float32)
    h = (h1 * jax.nn.sigmoid(h1)) * h3
    ye = jnp.dot(h.astype(jnp.bfloat16), w2_ref[0].astype(jnp.bfloat16),
                 preferred_element_type=jnp.float32)
    yall_ref[pl.ds(s * E, E), :] = ye

    xt = xt_ref[...]
    lg_ref[...] = jax.lax.dot_general(
        wg_ref[...], xt, (((1,), (1,)), ((), ())),
        preferred_element_type=jnp.float32,
    )

    xt16 = xt.astype(jnp.bfloat16)
    g1 = jax.lax.dot_general(
        xt16, sw1_ref[...].astype(jnp.bfloat16), (((1,), (1,)), ((), ())),
        preferred_element_type=jnp.float32,
    )
    g3 = jax.lax.dot_general(
        xt16, sw3_ref[...].astype(jnp.bfloat16), (((1,), (1,)), ((), ())),
        preferred_element_type=jnp.float32,
    )
    hs = (g1 * jax.nn.sigmoid(g1)) * g3
    out_ref[...] = jax.lax.dot_general(
        hs.astype(jnp.bfloat16), sw2_ref[...].astype(jnp.bfloat16),
        (((1,), (1,)), ((), ())), preferred_element_type=jnp.float32,
    )


def _sc_router(lg_hbm, c_hbm, lgv, tv, wv, vecv, cntbuf, crows, cfin,
               cnt_sh, c_sh, *, T, E, NS, L):
    cid = lax.axis_index("c")
    tid = lax.axis_index("s")
    per = T // NS
    ngrp = per // L
    base = tid * per

    @pl.when(cid == 0)
    def _():
        for e in range(E):
            pltpu.sync_copy(lg_hbm.at[pl.ds(e * T + base, per)],
                            lgv.at[pl.ds(e * per, per)])
        iota = lax.broadcasted_iota(jnp.int32, (L,), 0)
        zero = jnp.zeros((L,), jnp.float32)
        cnt_acc = [zero for _ in range(E)]
        for g in range(ngrp):
            ls = [lgv[pl.ds(e * per + g * L, L)] for e in range(E)]
            m = ls[0]
            for e in range(1, E):
                m = jnp.maximum(m, ls[e])
            denom = zero
            for e in range(E):
                denom = denom + jnp.exp(ls[e] - m)
            best = ls[0]
            bi = jnp.zeros((L,), jnp.int32)
            for e in range(1, E):
                gt = ls[e] > best
                bi = jnp.where(gt, e, bi)
                best = jnp.where(gt, ls[e], best)
            w = jnp.exp(best - m) / denom
            tv[pl.ds(g * L, L)] = bi
            wv[pl.ds(g * L, L)] = w
            for e in range(E):
                cnt_acc[e] = cnt_acc[e] + jnp.where(bi == e, 1.0, 0.0)
        for e in range(E):
            cntbuf[pl.ds(e * L, L)] = cnt_acc[e]
        for e in range(E, L):
            cntbuf[pl.ds(e * L, L)] = zero
        cnts = zero
        for l in range(L):
            cnts = cnts + plsc.load_gather(cntbuf, [iota * L + l])
        vecv[...] = cnts
        pltpu.sync_copy(vecv, cnt_sh.at[pl.ds(tid * L, L)])
        plsc.subcore_barrier()
        pltpu.sync_copy(cnt_sh, cfin.at[pl.ds(0, NS * L)])
        tot = cfin[pl.ds(0, L)]
        for t2 in range(1, NS):
            tot = tot + cfin[pl.ds(t2 * L, L)]
        cntbuf[pl.ds(L, L)] = tot
        offb = []
        acc = plsc.load_gather(cntbuf, [jnp.full((L,), L, jnp.int32)])
        offb.append(acc)
        for e in range(1, E):
            acc = acc + plsc.load_gather(
                cntbuf, [jnp.full((L,), L + e, jnp.int32)])
            offb.append(acc)
        for j in range(E * E):
            crows[pl.ds(j * L, L)] = zero
        for g in range(ngrp):
            bi = tv[pl.ds(g * L, L)]
            w = wv[pl.ds(g * L, L)]
            posf = (base + g * L + iota).astype(jnp.float32)
            segmask = [posf >= offb[e] for e in range(E - 1)]
            for r in range(E):
                wr = jnp.where(bi == r, w, 0.0)
                lo = wr
                for e in range(E - 1):
                    hi = jnp.where(segmask[e], wr, 0.0)
                    seg_e = lo - hi
                    idx = r * E + e
                    crows[pl.ds(idx * L, L)] = crows[pl.ds(idx * L, L)] + seg_e
                    lo = hi
                idx = r * E + (E - 1)
                crows[pl.ds(idx * L, L)] = crows[pl.ds(idx * L, L)] + lo
        nblk = (E * E) // L
        for b in range(nblk):
            acc = zero
            for l in range(L):
                acc = acc + plsc.load_gather(crows, [iota * L + (b * L * L + l)])
            vecv[...] = acc
            pltpu.sync_copy(vecv, c_sh.at[pl.ds(tid * E * E + b * L, L)])
        plsc.subcore_barrier()

        @pl.when(tid == 0)
        def _():
            pltpu.sync_copy(c_sh, cfin)
            for b in range(nblk):
                acc = cfin[pl.ds(b * L, L)]
                for t2 in range(1, NS):
                    acc = acc + cfin[pl.ds(t2 * E * E + b * L, L)]
                vecv[...] = acc
                pltpu.sync_copy(vecv, c_hbm.at[pl.ds(b * L, L)])


def _combine_kernel(c_ref, yall_ref, sh_ref, o_ref, *, E):
    c = c_ref[...]
    acc = sh_ref[...]
    for e in range(E):
        acc = acc + yall_ref[pl.ds(e * E, E), :] * c[:, e:e + 1]
    o_ref[...] = acc


def kernel(x, w_gate, w1, w2, w3, sw1, sw2, sw3):
    bs, slen, dim = x.shape
    xf = x.reshape(-1, dim)
    T = xf.shape[0]
    E = w_gate.shape[0]
    H = w1.shape[2]
    TILE = 256
    STEPS = T // TILE
    assert STEPS == E

    x8 = xf[:E]
    out_sh, lgT, yall = pl.pallas_call(
        functools.partial(_main_kernel, E=E, TILE=TILE),
        grid=(STEPS,),
        in_specs=[
            pl.BlockSpec((E, dim), lambda s: (0, 0)),
            pl.BlockSpec((E, dim), lambda s: (0, 0)),
            pl.BlockSpec((TILE, dim), lambda s: (s, 0)),
            pl.BlockSpec((1, dim, H), lambda s: (s, 0, 0)),
            pl.BlockSpec((1, H, dim), lambda s: (s, 0, 0)),
            pl.BlockSpec((1, dim, H), lambda s: (s, 0, 0)),
            pl.BlockSpec((H, dim), lambda s: (0, 0)),
            pl.BlockSpec((dim, H), lambda s: (0, 0)),
            pl.BlockSpec((H, dim), lambda s: (0, 0)),
        ],
        out_specs=[
            pl.BlockSpec((TILE, dim), lambda s: (s, 0)),
            pl.BlockSpec((E, TILE), lambda s: (0, s)),
            pl.BlockSpec((E * E, dim), lambda s: (0, 0)),
        ],
        out_shape=[
            jax.ShapeDtypeStruct((T, dim), jnp.float32),
            jax.ShapeDtypeStruct((E, T), jnp.float32),
            jax.ShapeDtypeStruct((E * E, dim), jnp.float32),
        ],
    )(x8, w_gate, xf, w1, w2, w3, sw1, sw2, sw3)

    info = plsc.get_sparse_core_info()
    NS, L = info.num_subcores, info.num_lanes
    sc_router = functools.partial(
        pl.kernel,
        mesh=plsc.VectorSubcoreMesh(core_axis_name="c", subcore_axis_name="s"),
        out_type=jax.ShapeDtypeStruct((E * E,), jnp.float32),
        compiler_params=pltpu.CompilerParams(needs_layout_passes=False),
        scratch_types=[
            pltpu.VMEM((E * (T // NS),), jnp.float32),
            pltpu.VMEM((T // NS,), jnp.int32),
            pltpu.VMEM((T // NS,), jnp.float32),
            pltpu.VMEM((L,), jnp.float32),
            pltpu.VMEM((L * L,), jnp.float32),
            pltpu.VMEM((E * E * L,), jnp.float32),
            pltpu.VMEM((NS * E * E,), jnp.float32),
            pltpu.VMEM_SHARED((NS * L,), jnp.float32),
            pltpu.VMEM_SHARED((NS * E * E,), jnp.float32),
        ],
    )(functools.partial(_sc_router, T=T, E=E, NS=NS, L=L))
    c64 = sc_router(lgT.reshape(E * T))
    c = c64.reshape(E, E)

    out = pl.pallas_call(
        functools.partial(_combine_kernel, E=E),
        grid=(1,),
        in_specs=[
            pl.BlockSpec((E, E), lambda i: (0, 0)),
            pl.BlockSpec((E * E, dim), lambda i: (0, 0)),
            pl.BlockSpec((E, dim), lambda i: (0, 0)),
        ],
        out_specs=pl.BlockSpec((E, dim), lambda i: (0, 0)),
        out_shape=jax.ShapeDtypeStruct((T, dim), jnp.float32),
        input_output_aliases={2: 0},
    )(c, yall, out_sh)

    return out.reshape(bs, slen, dim).astype(x.dtype)


# speedup vs baseline: 1.0340x; 1.0340x over previous
import functools

import jax
import jax.numpy as jnp
from jax import lax
from jax.experimental import pallas as pl
from jax.experimental.pallas import tpu as pltpu
from jax.experimental.pallas import tpu_sc as plsc


def _logits_kernel(xt_ref, wg_ref, lg_ref):
    lg_ref[...] = jax.lax.dot_general(
        xt_ref[...], wg_ref[...], (((1,), (1,)), ((), ())),
        preferred_element_type=jnp.float32,
    )


def _main_kernel(x8_ref, xt_ref, w1_ref, w2_ref, w3_ref,
                 sw1_ref, sw2_ref, sw3_ref, out_ref, yall_ref,
                 *, E, TILE):
    s = pl.program_id(0)
    x8 = x8_ref[...].astype(jnp.bfloat16)
    h1 = jnp.dot(x8, w1_ref[0].astype(jnp.bfloat16),
                 preferred_element_type=jnp.float32)
    h3 = jnp.dot(x8, w3_ref[0].astype(jnp.bfloat16),
                 preferred_element_type=jnp.float32)
    h = (h1 * jax.nn.sigmoid(h1)) * h3
    ye = jnp.dot(h.astype(jnp.bfloat16), w2_ref[0].astype(jnp.bfloat16),
                 preferred_element_type=jnp.float32)
    yall_ref[pl.ds(s * E, E), :] = ye

    xt = xt_ref[...]
    xt16 = xt.astype(jnp.bfloat16)
    g1 = jax.lax.dot_general(
        xt16, sw1_ref[...].astype(jnp.bfloat16), (((1,), (1,)), ((), ())),
        preferred_element_type=jnp.float32,
    )
    g3 = jax.lax.dot_general(
        xt16, sw3_ref[...].astype(jnp.bfloat16), (((1,), (1,)), ((), ())),
        preferred_element_type=jnp.float32,
    )
    hs = (g1 * jax.nn.sigmoid(g1)) * g3
    out_ref[...] = jax.lax.dot_general(
        hs.astype(jnp.bfloat16), sw2_ref[...].astype(jnp.bfloat16),
        (((1,), (1,)), ((), ())), preferred_element_type=jnp.float32,
    )


def _sc_router(lg_hbm, c_hbm, lgv, tv, wv, vecv, cntbuf, crows, cfin,
               cnt_sh, c_sh, *, T, E, NS, L):
    cid = lax.axis_index("c")
    tid = lax.axis_index("s")
    per = T // NS
    ngrp = per // L
    base = tid * per

    @pl.when(cid == 0)
    def _():
        pltpu.sync_copy(lg_hbm.at[pl.ds(base * E, per * E)], lgv)
        iota = lax.broadcasted_iota(jnp.int32, (L,), 0)
        zero = jnp.zeros((L,), jnp.float32)
        cnt_acc = [zero for _ in range(E)]
        for g in range(ngrp):
            ls = [plsc.load_gather(lgv, [iota * E + (g * L * E + e)])
                  for e in range(E)]
            m = ls[0]
            for e in range(1, E):
                m = jnp.maximum(m, ls[e])
            denom = zero
            for e in range(E):
                denom = denom + jnp.exp(ls[e] - m)
            best = ls[0]
            bi = jnp.zeros((L,), jnp.int32)
            for e in range(1, E):
                gt = ls[e] > best
                bi = jnp.where(gt, e, bi)
                best = jnp.where(gt, ls[e], best)
            w = jnp.exp(best - m) / denom
            tv[pl.ds(g * L, L)] = bi
            wv[pl.ds(g * L, L)] = w
            for e in range(E):
                cnt_acc[e] = cnt_acc[e] + jnp.where(bi == e, 1.0, 0.0)
        for e in range(E):
            cntbuf[pl.ds(e * L, L)] = cnt_acc[e]
        for e in range(E, L):
            cntbuf[pl.ds(e * L, L)] = zero
        cnts = zero
        for l in range(L):
            cnts = cnts + plsc.load_gather(cntbuf, [iota * L + l])
        vecv[...] = cnts
        pltpu.sync_copy(vecv, cnt_sh.at[pl.ds(tid * L, L)])
        plsc.subcore_barrier()
        pltpu.sync_copy(cnt_sh, cfin.at[pl.ds(0, NS * L)])
        tot = cfin[pl.ds(0, L)]
        for t2 in range(1, NS):
            tot = tot + cfin[pl.ds(t2 * L, L)]
        cntbuf[pl.ds(L, L)] = tot
        offb = []
        acc = plsc.load_gather(cntbuf, [jnp.full((L,), L, jnp.int32)])
        offb.append(acc)
        for e in range(1, E):
            acc = acc + plsc.load_gather(
                cntbuf, [jnp.full((L,), L + e, jnp.int32)])
            offb.append(acc)
        for j in range(E * E):
            crows[pl.ds(j * L, L)] = zero
        for g in range(ngrp):
            bi = tv[pl.ds(g * L, L)]
            w = wv[pl.ds(g * L, L)]
            posf = (base + g * L + iota).astype(jnp.float32)
            segmask = [posf >= offb[e] for e in range(E - 1)]
            for r in range(E):
                wr = jnp.where(bi == r, w, 0.0)
                lo = wr
                for e in range(E - 1):
                    hi = jnp.where(segmask[e], wr, 0.0)
                    seg_e = lo - hi
                    idx = r * E + e
                    crows[pl.ds(idx * L, L)] = crows[pl.ds(idx * L, L)] + seg_e
                    lo = hi
                idx = r * E + (E - 1)
                crows[pl.ds(idx * L, L)] = crows[pl.ds(idx * L, L)] + lo
        nblk = (E * E) // L
        for b in range(nblk):
            acc = zero
            for l in range(L):
                acc = acc + plsc.load_gather(crows, [iota * L + (b * L * L + l)])
            vecv[...] = acc
            pltpu.sync_copy(vecv, c_sh.at[pl.ds(tid * E * E + b * L, L)])
        plsc.subcore_barrier()

        @pl.when(tid == 0)
        def _():
            pltpu.sync_copy(c_sh, cfin)
            for b in range(nblk):
                acc = cfin[pl.ds(b * L, L)]
                for t2 in range(1, NS):
                    acc = acc + cfin[pl.ds(t2 * E * E + b * L, L)]
                vecv[...] = acc
                pltpu.sync_copy(vecv, c_hbm.at[pl.ds(b * L, L)])


def _combine_kernel(c_ref, yall_ref, sh_ref, o_ref, *, E):
    c = c_ref[...]
    acc = sh_ref[...]
    for e in range(E):
        acc = acc + yall_ref[pl.ds(e * E, E), :] * c[:, e:e + 1]
    o_ref[...] = acc


def kernel(x, w_gate, w1, w2, w3, sw1, sw2, sw3):
    bs, slen, dim = x.shape
    xf = x.reshape(-1, dim)
    T = xf.shape[0]
    E = w_gate.shape[0]
    H = w1.shape[2]
    TILE = 256
    STEPS = T // TILE
    assert STEPS == E

    lg = pl.pallas_call(
        _logits_kernel,
        grid=(STEPS,),
        in_specs=[
            pl.BlockSpec((TILE, dim), lambda s: (s, 0)),
            pl.BlockSpec((E, dim), lambda s: (0, 0)),
        ],
        out_specs=pl.BlockSpec((TILE, E), lambda s: (s, 0)),
        out_shape=jax.ShapeDtypeStruct((T, E), jnp.float32),
    )(xf, w_gate)

    x8 = xf[:E]
    out_sh, yall = pl.pallas_call(
        functools.partial(_main_kernel, E=E, TILE=TILE),
        grid=(STEPS,),
        in_specs=[
            pl.BlockSpec((E, dim), lambda s: (0, 0)),
            pl.BlockSpec((TILE, dim), lambda s: (s, 0)),
            pl.BlockSpec((1, dim, H), lambda s: (s, 0, 0)),
            pl.BlockSpec((1, H, dim), lambda s: (s, 0, 0)),
            pl.BlockSpec((1, dim, H), lambda s: (s, 0, 0)),
            pl.BlockSpec((H, dim), lambda s: (0, 0)),
            pl.BlockSpec((dim, H), lambda s: (0, 0)),
            pl.BlockSpec((H, dim), lambda s: (0, 0)),
        ],
        out_specs=[
            pl.BlockSpec((TILE, dim), lambda s: (s, 0)),
            pl.BlockSpec((E * E, dim), lambda s: (0, 0)),
        ],
        out_shape=[
            jax.ShapeDtypeStruct((T, dim), jnp.float32),
            jax.ShapeDtypeStruct((E * E, dim), jnp.float32),
        ],
    )(x8, xf, w1, w2, w3, sw1, sw2, sw3)

    info = plsc.get_sparse_core_info()
    NS, L = info.num_subcores, info.num_lanes
    sc_router = functools.partial(
        pl.kernel,
        mesh=plsc.VectorSubcoreMesh(core_axis_name="c", subcore_axis_name="s"),
        out_type=jax.ShapeDtypeStruct((E * E,), jnp.float32),
        compiler_params=pltpu.CompilerParams(needs_layout_passes=False),
        scratch_types=[
            pltpu.VMEM((E * (T // NS),), jnp.float32),
            pltpu.VMEM((T // NS,), jnp.int32),
            pltpu.VMEM((T // NS,), jnp.float32),
            pltpu.VMEM((L,), jnp.float32),
            pltpu.VMEM((L * L,), jnp.float32),
            pltpu.VMEM((E * E * L,), jnp.float32),
            pltpu.VMEM((NS * E * E,), jnp.float32),
            pltpu.VMEM_SHARED((NS * L,), jnp.float32),
            pltpu.VMEM_SHARED((NS * E * E,), jnp.float32),
        ],
    )(functools.partial(_sc_router, T=T, E=E, NS=NS, L=L))
    c64 = sc_router(lg.reshape(T * E))
    c = c64.reshape(E, E)

    out = pl.pallas_call(
        functools.partial(_combine_kernel, E=E),
        grid=(1,),
        in_specs=[
            pl.BlockSpec((E, E), lambda i: (0, 0)),
            pl.BlockSpec((E * E, dim), lambda i: (0, 0)),
            pl.BlockSpec((E, dim), lambda i: (0, 0)),
        ],
        out_specs=pl.BlockSpec((E, dim), lambda i: (0, 0)),
        out_shape=jax.ShapeDtypeStruct((T, dim), jnp.float32),
        input_output_aliases={2: 0},
    )(c, yall, out_sh)

    return out.reshape(bs, slen, dim).astype(x.dtype)
